# Initial kernel scaffold; baseline (speedup 1.0000x reference)
import jax
import jax.numpy as jnp
from jax.experimental import pallas as pl


def kernel(x, y):
    raise NotImplementedError("write your pallas kernel here")


# speedup vs baseline: 1026.6027x; 1026.6027x over previous
import jax
import jax.numpy as jnp
from jax.experimental import pallas as pl
from jax.experimental.pallas import tpu as pltpu

_INF = 99999.0
_N = 512
_L = 640
_DMAX = 2 * _N
_RB = 64
_CB = 128
_GR = 9
_GC = 5
_BROWS = 1216


def _dtw_body(xs_ref, yp_ref, out_ref, b_ref):
    ir = pl.program_id(0)
    ic = pl.program_id(1)

    @pl.when(jnp.logical_and(ir == 0, ic == 0))
    def _dp():
        j = jax.lax.broadcasted_iota(jnp.int32, (1, _L), 1)
        yp = yp_ref[:, :]
        e0 = jnp.where(j == 0, 0.0, _INF).astype(jnp.float32)
        for k in range(_GC):
            b_ref[k, pl.ds(0, 1), :] = e0[:, k * 128:(k + 1) * 128]

        def body(d, carry):
            prev, sprev2, xs = carry
            xs = pltpu.roll(xs, 1, 1)
            sprev = pltpu.roll(prev, 1, 1)
            cost = (xs - yp) ** 2
            m = jnp.minimum(jnp.minimum(prev, sprev), sprev2)
            raw = cost + m
            i_idx = d - j
            valid = (j >= 1) & (j <= _N) & (i_idx >= 1) & (i_idx <= _N)
            e = jnp.where(valid, raw, _INF)
            for k in range(_GC):
                b_ref[k, pl.ds(d, 1), :] = e[:, k * 128:(k + 1) * 128]
            return (e, sprev, xs)

        init = (e0, jnp.full((1, _L), _INF, jnp.float32), xs_ref[:, :])
        jax.lax.fori_loop(1, _DMAX + 1, body, init)

    start = ir * _RB + ic * _CB
    v = b_ref[ic, pl.ds(start, 192), :]
    lane = jax.lax.broadcasted_iota(jnp.int32, (192, 128), 1)
    for s in (64, 32, 16, 8, 4, 2, 1):
        rolled = pltpu.roll(v, 192 - s, 0)
        v = jnp.where((lane & s) != 0, rolled, v)
    out_ref[:, :] = jnp.sqrt(v[:_RB, :])


def kernel(x, y):
    x = x.astype(jnp.float32)
    y = y.astype(jnp.float32)
    xcirc = jnp.concatenate([x, jnp.zeros((_L - _N,), jnp.float32)])
    xs1 = jnp.flip(xcirc)[None, :]
    yp = jnp.concatenate(
        [jnp.zeros((1,), jnp.float32), y, jnp.zeros((_L - _N - 1,), jnp.float32)]
    )[None, :]

    out = pl.pallas_call(
        _dtw_body,
        grid=(_GR, _GC),
        in_specs=[
            pl.BlockSpec((1, _L), lambda i, j: (0, 0)),
            pl.BlockSpec((1, _L), lambda i, j: (0, 0)),
        ],
        out_specs=pl.BlockSpec((_RB, _CB), lambda i, j: (i, j)),
        out_shape=jax.ShapeDtypeStruct((_N + 1, _N + 1), jnp.float32),
        scratch_shapes=[pltpu.VMEM((_GC, _BROWS, 128), jnp.float32)],
    )(xs1, yp)
    return out


# speedup vs baseline: 1239.9442x; 1.2078x over previous
import jax
import jax.numpy as jnp
from jax.experimental import pallas as pl
from jax.experimental.pallas import tpu as pltpu

_INF = 99999.0
_N = 512
_DMAX = 2 * _N
_RB = 64
_GR = 9
_BROWS = 1216


def _dtw_body(xs_ref, yp_ref, out_ref, b_ref):
    ir = pl.program_id(0)

    @pl.when(ir == 0)
    def _dp():
        s_io = jax.lax.broadcasted_iota(jnp.int32, (8, 128), 0)
        l_io = jax.lax.broadcasted_iota(jnp.int32, (8, 128), 1)
        j = s_io * 128 + l_io
        yp = yp_ref[:, :]
        e0 = jnp.where(j == 0, 0.0, _INF).astype(jnp.float32)
        b_ref[pl.ds(0, 1), :, :] = e0[None]

        def shift1(v):
            a = pltpu.roll(v, 1, 1)
            b = pltpu.roll(a, 1, 0)
            return jnp.where(l_io == 0, b, a)

        def body(d, carry):
            prev, sprev2, xs = carry
            xs = shift1(xs)
            sprev = shift1(prev)
            cost = (xs - yp) ** 2
            raw = cost + jnp.minimum(jnp.minimum(prev, sprev), sprev2)
            i_idx = d - j
            valid = (j >= 1) & (j <= _N) & (i_idx >= 1) & (i_idx <= _N)
            e = jnp.where(valid, raw, _INF)
            b_ref[pl.ds(d, 1), :, :] = e[None]
            return (e, sprev, xs)

        init = (e0, jnp.full((8, 128), _INF, jnp.float32), xs_ref[:, :])
        jax.lax.fori_loop(1, _DMAX + 1, body, init, unroll=2)

    r0 = ir * _RB
    lane = jax.lax.broadcasted_iota(jnp.int32, (192, 128), 1)
    for k in range(5):
        v = b_ref[pl.ds(r0 + k * 128, 192), k, :]
        for s in (64, 32, 16, 8, 4, 2, 1):
            rolled = pltpu.roll(v, 192 - s, 0)
            v = jnp.where((lane & s) != 0, rolled, v)
        out_ref[:, k * 128:(k + 1) * 128] = jnp.sqrt(v[:_RB, :])


def kernel(x, y):
    x = x.astype(jnp.float32)
    y = y.astype(jnp.float32)
    xcirc = jnp.concatenate([x, jnp.zeros((_N,), jnp.float32)])
    xs0 = jnp.flip(xcirc).reshape(8, 128)
    yp = jnp.concatenate(
        [jnp.zeros((1,), jnp.float32), y, jnp.zeros((_N - 1,), jnp.float32)]
    ).reshape(8, 128)

    out = pl.pallas_call(
        _dtw_body,
        grid=(_GR,),
        in_specs=[
            pl.BlockSpec((8, 128), lambda i: (0, 0)),
            pl.BlockSpec((8, 128), lambda i: (0, 0)),
        ],
        out_specs=pl.BlockSpec((_RB, 640), lambda i: (i, 0)),
        out_shape=jax.ShapeDtypeStruct((_N + 1, _N + 1), jnp.float32),
        scratch_shapes=[pltpu.VMEM((_BROWS, 8, 128), jnp.float32)],
    )(xs0, yp)
    return out


# speedup vs baseline: 1248.0368x; 1.0065x over previous
import jax
import jax.numpy as jnp
from jax.experimental import pallas as pl
from jax.experimental.pallas import tpu as pltpu

_INF = 99999.0
_N = 512
_DMAX = 2 * _N
_RB = 64
_GR = 9
_BROWS = 1216


def _dtw_body(xs_ref, yp_ref, out_ref, b_ref):
    ir = pl.program_id(0)

    @pl.when(ir == 0)
    def _dp():
        s_io = jax.lax.broadcasted_iota(jnp.int32, (8, 128), 0)
        l_io = jax.lax.broadcasted_iota(jnp.int32, (8, 128), 1)
        j = s_io * 128 + l_io
        yp = yp_ref[:, :]
        e0 = jnp.where(j == 0, 0.0, _INF).astype(jnp.float32)
        b_ref[pl.ds(0, 1), :, :] = e0[None]

        def shift1(v):
            a = pltpu.roll(v, 1, 1)
            b = pltpu.roll(a, 1, 0)
            return jnp.where(l_io == 0, b, a)

        j_ge1 = j >= 1

        def body(d, carry):
            prev, sprev2, xs = carry
            xs = shift1(xs)
            sprev = shift1(prev)
            cost = (xs - yp) ** 2
            raw = cost + jnp.minimum(jnp.minimum(prev, sprev), sprev2)
            e = jnp.where(j_ge1 & (j < d), raw, _INF)
            b_ref[pl.ds(d, 1), :, :] = e[None]
            return (e, sprev, xs)

        init = (e0, jnp.full((8, 128), _INF, jnp.float32), xs_ref[:, :])
        jax.lax.fori_loop(1, _DMAX + 1, body, init, unroll=8)

    r0 = ir * _RB
    lane = jax.lax.broadcasted_iota(jnp.int32, (192, 128), 1)
    for k in range(5):
        v = b_ref[pl.ds(r0 + k * 128, 192), k, :]
        for s in (64, 32, 16, 8, 4, 2, 1):
            rolled = pltpu.roll(v, 192 - s, 0)
            v = jnp.where((lane & s) != 0, rolled, v)
        out_ref[:, k * 128:(k + 1) * 128] = jnp.sqrt(v[:_RB, :])


def kernel(x, y):
    x = x.astype(jnp.float32)
    y = y.astype(jnp.float32)
    xcirc = jnp.concatenate([x, jnp.zeros((_N,), jnp.float32)])
    xs0 = jnp.flip(xcirc).reshape(8, 128)
    yp = jnp.concatenate(
        [jnp.zeros((1,), jnp.float32), y, jnp.zeros((_N - 1,), jnp.float32)]
    ).reshape(8, 128)

    out = pl.pallas_call(
        _dtw_body,
        grid=(_GR,),
        in_specs=[
            pl.BlockSpec((8, 128), lambda i: (0, 0)),
            pl.BlockSpec((8, 128), lambda i: (0, 0)),
        ],
        out_specs=pl.BlockSpec((_RB, 640), lambda i: (i, 0)),
        out_shape=jax.ShapeDtypeStruct((_N + 1, _N + 1), jnp.float32),
        scratch_shapes=[pltpu.VMEM((_BROWS, 8, 128), jnp.float32)],
    )(xs0, yp)
    return out


# speedup vs baseline: 4094.3516x; 3.2806x over previous
import jax
import jax.numpy as jnp
from jax.experimental import pallas as pl
from jax.experimental.pallas import tpu as pltpu

_INF = 99999.0
_N = 512
_DMAX = 2 * _N
_K = 8
_RB = 64
_GR = 9
_BROWS = 1216


def _dtw_body(xs_ref, yp_ref, out_ref, b_ref):
    ir = pl.program_id(0)

    @pl.when(ir == 0)
    def _dp():
        s_io = jax.lax.broadcasted_iota(jnp.int32, (8, 128), 0)
        l_io = jax.lax.broadcasted_iota(jnp.int32, (8, 128), 1)
        j = s_io * 128 + l_io
        yp = yp_ref[:, :]
        e0 = jnp.where(j == 0, 0.0, _INF).astype(jnp.float32)
        b_ref[pl.ds(0, 1), :, :] = e0[None]

        def sigma(v, a):
            r = pltpu.roll(v, a, 1)
            rr = pltpu.roll(r, 1, 0)
            return jnp.where(l_io < a, rr, r)

        yps = [yp] + [sigma(yp, a) for a in range(1, _K)]
        jus = [(j - (a + 1)).astype(jnp.uint32) for a in range(_K)]

        def body(it, carry):
            P, Q, X = carry
            d0 = it * _K
            PA = [P] + [sigma(P, a) for a in range(1, _K + 1)]
            QA = [None] + [sigma(Q, a) for a in range(1, _K + 1)]
            XA = [X] + [sigma(X, a) for a in range(1, _K + 1)]
            vprev2, vprev = QA, PA
            for t in range(1, _K + 1):
                vt = []
                for a in range(_K - t + 1):
                    cost = (XA[t + a] - yps[a]) ** 2
                    raw = cost + jnp.minimum(
                        jnp.minimum(vprev[a], vprev[a + 1]), vprev2[a + 1]
                    )
                    e = jnp.where(jus[a] < jnp.uint32(d0 + t - 1), raw, _INF)
                    vt.append(e)
                b_ref[pl.ds(d0 + t, 1), :, :] = vt[0][None]
                vprev2, vprev = vprev, vt
            return (vprev[0], vprev2[0], XA[_K])

        init = (e0, jnp.full((8, 128), _INF, jnp.float32), xs_ref[:, :])
        jax.lax.fori_loop(0, _DMAX // _K, body, init)

    r0 = ir * _RB
    lane = jax.lax.broadcasted_iota(jnp.int32, (192, 128), 1)
    for k in range(5):
        v = b_ref[pl.ds(r0 + k * 128, 192), k, :]
        for s in (64, 32, 16, 8, 4, 2, 1):
            rolled = pltpu.roll(v, 192 - s, 0)
            v = jnp.where((lane & s) != 0, rolled, v)
        out_ref[:, k * 128:(k + 1) * 128] = jnp.sqrt(v[:_RB, :])


def kernel(x, y):
    x = x.astype(jnp.float32)
    y = y.astype(jnp.float32)
    xcirc = jnp.concatenate([x, jnp.zeros((_N,), jnp.float32)])
    xs0 = jnp.flip(xcirc).reshape(8, 128)
    yp = jnp.concatenate(
        [jnp.zeros((1,), jnp.float32), y, jnp.zeros((_N - 1,), jnp.float32)]
    ).reshape(8, 128)

    out = pl.pallas_call(
        _dtw_body,
        grid=(_GR,),
        in_specs=[
            pl.BlockSpec((8, 128), lambda i: (0, 0)),
            pl.BlockSpec((8, 128), lambda i: (0, 0)),
        ],
        out_specs=pl.BlockSpec((_RB, 640), lambda i: (i, 0)),
        out_shape=jax.ShapeDtypeStruct((_N + 1, _N + 1), jnp.float32),
        scratch_shapes=[pltpu.VMEM((_BROWS, 8, 128), jnp.float32)],
    )(xs0, yp)
    return out
